# Initial kernel scaffold; baseline (speedup 1.0000x reference)
import jax
import jax.numpy as jnp
from jax.experimental import pallas as pl


def kernel(query, global_patterns, classA_patterns, classB_patterns, W1, b1, W2, b2):
    raise NotImplementedError("write your pallas kernel here")


# speedup vs baseline: 1.5528x; 1.5528x over previous
import functools

import jax
import jax.numpy as jnp
from jax.experimental import pallas as pl

_Q = 1024
_D = 512
_BQ = 256

def _retrieve(q, p):
    pb = p.astype(jnp.bfloat16)
    sim = jax.lax.dot_general(
        q.astype(jnp.bfloat16), pb, (((1,), (1,)), ((), ())),
        preferred_element_type=jnp.float32)
    m = jnp.max(sim, axis=-1, keepdims=True)
    e = jnp.exp(sim - m)
    s = jnp.sum(e, axis=-1, keepdims=True)
    attn = (e / s).astype(jnp.bfloat16)
    return jax.lax.dot_general(
        attn, pb, (((1,), (0,)), ((), ())),
        preferred_element_type=jnp.float32)


def _body(q_ref, pg_ref, pa_ref, pb_ref, w1_ref, b1_ref, w2t_ref, b2_ref,
          o_ref):
    q = q_ref[...]
    rg = _retrieve(q, pg_ref[...])
    ra = _retrieve(q, pa_ref[...])
    rb = _retrieve(q, pb_ref[...])
    cr = 0.5 * (ra + rb)

    comb = jnp.concatenate([cr, rg], axis=-1)
    h = jax.lax.dot_general(
        comb.astype(jnp.bfloat16), w1_ref[...].astype(jnp.bfloat16),
        (((1,), (0,)), ((), ())),
        preferred_element_type=jnp.float32) + b1_ref[...]
    h = 0.5 * h * (1.0 + jax.lax.erf(h * 0.7071067811865476))
    logit = jnp.sum(h * w2t_ref[...], axis=-1, keepdims=True) + b2_ref[...]
    gate = jax.nn.sigmoid(logit)
    o_ref[...] = gate * cr + (1.0 - gate) * rg


@functools.partial(jax.jit, static_argnames=())
def kernel(query, global_patterns, classA_patterns, classB_patterns,
           W1, b1, W2, b2):
    kg = global_patterns.shape[0]
    kc = classA_patterns.shape[0]
    grid = (_Q // _BQ,)
    out = pl.pallas_call(
        _body,
        grid=grid,
        in_specs=[
            pl.BlockSpec((_BQ, _D), lambda i: (i, 0)),
            pl.BlockSpec((kg, _D), lambda i: (0, 0)),
            pl.BlockSpec((kc, _D), lambda i: (0, 0)),
            pl.BlockSpec((kc, _D), lambda i: (0, 0)),
            pl.BlockSpec((2 * _D, 64), lambda i: (0, 0)),
            pl.BlockSpec((1, 64), lambda i: (0, 0)),
            pl.BlockSpec((1, 64), lambda i: (0, 0)),
            pl.BlockSpec((1, 1), lambda i: (0, 0)),
        ],
        out_specs=pl.BlockSpec((_BQ, _D), lambda i: (i, 0)),
        out_shape=jax.ShapeDtypeStruct((_Q, _D), jnp.float32),
    )(query, global_patterns, classA_patterns, classB_patterns,
      W1, b1.reshape(1, 64), W2.reshape(1, 64), b2.reshape(1, 1))
    return out
